# baseline (device time: 174738 ns/iter reference)
import jax
import jax.numpy as jnp
from jax import lax
from jax.experimental import pallas as pl
from jax.experimental.pallas import tpu as pltpu

N_DEV = 16
SQ = 1024
SKV = 1024
D_MODEL = 1024
HEADS_PER_SHARD = 8
DH = 128
WINDOW = 128
SCALE = 0.08838834764831843
CHUNK = SQ // N_DEV
N_STEPS = N_DEV - 1


def _body(x_ref, wq_ref, k_ref, v_ref, wo_ref, out_ref,
          q_ref, ctx_ref, acc_ref, rs_buf,
          rs_send_sems, rs_recv_sems, ag_send_sems, ag_recv_sems):
    my = lax.axis_index("i")
    left = (my - 1) % N_DEV
    right = (my + 1) % N_DEV

    q = lax.dot_general(
        x_ref[...], wq_ref[...], (((1,), (0,)), ((), ())),
        preferred_element_type=jnp.float32,
    )
    q_ref[...] = q.astype(jnp.bfloat16)

    rows = lax.broadcasted_iota(jnp.int32, (SQ, SKV), 0)
    cols = lax.broadcasted_iota(jnp.int32, (SQ, SKV), 1)
    mask = jnp.abs(rows - cols) <= WINDOW

    for h in range(HEADS_PER_SHARD):
        qh = q_ref[:, h * DH:(h + 1) * DH]
        scores = lax.dot_general(
            qh, k_ref[h], (((1,), (1,)), ((), ())),
            preferred_element_type=jnp.float32,
        ) * SCALE
        scores = jnp.where(mask, scores, -1e9)
        m = jnp.max(scores, axis=1, keepdims=True)
        e = jnp.exp(scores - m)
        s = jnp.sum(e, axis=1, keepdims=True)
        w = (e / s).astype(jnp.bfloat16)
        ctx = lax.dot_general(
            w, v_ref[h], (((1,), (0,)), ((), ())),
            preferred_element_type=jnp.float32,
        )
        ctx_ref[:, h * DH:(h + 1) * DH] = ctx.astype(jnp.bfloat16)

    acc_ref[...] = lax.dot_general(
        ctx_ref[...], wo_ref[...], (((1,), (0,)), ((), ())),
        preferred_element_type=jnp.float32,
    )

    bsem = pltpu.get_barrier_semaphore()
    for nbr in (left, right):
        pl.semaphore_signal(bsem, inc=1, device_id=(nbr,),
                            device_id_type=pl.DeviceIdType.MESH)
    pl.semaphore_wait(bsem, 2)

    for s in range(N_STEPS):
        cs = (my - s) % N_DEV
        cr = (my - s - 1) % N_DEV
        step = pltpu.make_async_remote_copy(
            src_ref=acc_ref.at[pl.ds(cs * CHUNK, CHUNK), :],
            dst_ref=rs_buf.at[s],
            send_sem=rs_send_sems.at[s],
            recv_sem=rs_recv_sems.at[s],
            device_id=(right,),
            device_id_type=pl.DeviceIdType.MESH,
        )
        step.start()
        step.wait_send()
        step.wait_recv()
        acc_ref[pl.ds(cr * CHUNK, CHUNK), :] = (
            acc_ref[pl.ds(cr * CHUNK, CHUNK), :] + rs_buf[s]
        )

    c_own = (my + 1) % N_DEV
    out_ref[pl.ds(c_own * CHUNK, CHUNK), :] = (
        acc_ref[pl.ds(c_own * CHUNK, CHUNK), :]
    )

    for s in range(N_STEPS):
        c_send = (my + 1 - s) % N_DEV
        c_recv = (my - s) % N_DEV
        send = pltpu.make_async_remote_copy(
            src_ref=out_ref.at[pl.ds(c_send * CHUNK, CHUNK), :],
            dst_ref=out_ref.at[pl.ds(c_send * CHUNK, CHUNK), :],
            send_sem=ag_send_sems.at[s],
            recv_sem=ag_recv_sems.at[s],
            device_id=(right,),
            device_id_type=pl.DeviceIdType.MESH,
        )
        send.start()
        send.wait_send()
        recv = pltpu.make_async_remote_copy(
            src_ref=out_ref.at[pl.ds(c_recv * CHUNK, CHUNK), :],
            dst_ref=out_ref.at[pl.ds(c_recv * CHUNK, CHUNK), :],
            send_sem=ag_send_sems.at[s],
            recv_sem=ag_recv_sems.at[s],
            device_id=(left,),
            device_id_type=pl.DeviceIdType.MESH,
        )
        recv.wait_recv()


def kernel(x, Wq, K_ext, V_ext, Wo):
    pos = lax.axis_index("i")
    xb = x[0].astype(jnp.bfloat16)
    wq = Wq.astype(jnp.bfloat16)
    wo = Wo.astype(jnp.bfloat16)
    kh = lax.dynamic_slice(
        K_ext, (0, 0, pos * HEADS_PER_SHARD, 0), (1, SKV, HEADS_PER_SHARD, DH)
    )[0]
    vh = lax.dynamic_slice(
        V_ext, (0, 0, pos * HEADS_PER_SHARD, 0), (1, SKV, HEADS_PER_SHARD, DH)
    )[0]
    kh = jnp.transpose(kh, (1, 0, 2)).astype(jnp.bfloat16)
    vh = jnp.transpose(vh, (1, 0, 2)).astype(jnp.bfloat16)

    out = pl.pallas_call(
        _body,
        out_shape=jax.ShapeDtypeStruct((SQ, D_MODEL), jnp.float32),
        in_specs=[pl.BlockSpec(memory_space=pltpu.VMEM)] * 5,
        out_specs=pl.BlockSpec(memory_space=pltpu.VMEM),
        scratch_shapes=[
            pltpu.VMEM((SQ, D_MODEL), jnp.bfloat16),
            pltpu.VMEM((SQ, D_MODEL), jnp.bfloat16),
            pltpu.VMEM((SQ, D_MODEL), jnp.float32),
            pltpu.VMEM((N_STEPS, CHUNK, D_MODEL), jnp.float32),
            pltpu.SemaphoreType.DMA((N_STEPS,)),
            pltpu.SemaphoreType.DMA((N_STEPS,)),
            pltpu.SemaphoreType.DMA((N_STEPS,)),
            pltpu.SemaphoreType.DMA((N_STEPS,)),
        ],
        compiler_params=pltpu.CompilerParams(collective_id=0),
    )(xb, wq, kh, vh, wo)
    return out.reshape(1, SQ, D_MODEL)


# device time: 94945 ns/iter; 1.8404x vs baseline; 1.8404x over previous
import jax
import jax.numpy as jnp
from jax import lax
from jax.experimental import pallas as pl
from jax.experimental.pallas import tpu as pltpu

N_DEV = 16
SQ = 1024
SKV = 1024
D_MODEL = 1024
HEADS_PER_SHARD = 8
DH = 128
WINDOW = 128
SCALE = 0.08838834764831843

RS_MASKS = (1, 2, 4, 8)
DB_MASKS = (8, 4, 2, 1)
CONTRIB = {1: 512, 2: 256, 4: 128, 8: 64}


def _body(x_ref, wq_ref, k_ref, v_ref, wo_ref, out_ref,
          q_ref, ctx_ref, acc_ref, sbuf_ref, rbuf_ref, gbuf_ref,
          rs_send_sems, rs_recv_sems, db_send_sems, db_recv_sems):
    my = lax.axis_index("i")

    q = lax.dot_general(
        x_ref[...], wq_ref[...], (((1,), (0,)), ((), ())),
        preferred_element_type=jnp.float32,
    )
    q_ref[...] = q.astype(jnp.bfloat16)

    rows = lax.broadcasted_iota(jnp.int32, (SQ, SKV), 0)
    cols = lax.broadcasted_iota(jnp.int32, (SQ, SKV), 1)
    mask = jnp.abs(rows - cols) <= WINDOW

    for h in range(HEADS_PER_SHARD):
        qh = q_ref[:, h * DH:(h + 1) * DH]
        scores = lax.dot_general(
            qh, k_ref[h], (((1,), (1,)), ((), ())),
            preferred_element_type=jnp.float32,
        ) * SCALE
        scores = jnp.where(mask, scores, -1e9)
        m = jnp.max(scores, axis=1, keepdims=True)
        e = jnp.exp(scores - m)
        s = jnp.sum(e, axis=1, keepdims=True)
        w = (e / s).astype(jnp.bfloat16)
        ctx = lax.dot_general(
            w, v_ref[h], (((1,), (0,)), ((), ())),
            preferred_element_type=jnp.float32,
        )
        ctx_ref[:, h * DH:(h + 1) * DH] = ctx.astype(jnp.bfloat16)

    acc_ref[...] = lax.dot_general(
        ctx_ref[...], wo_ref[...], (((1,), (0,)), ((), ())),
        preferred_element_type=jnp.float32,
    )

    bsem = pltpu.get_barrier_semaphore()
    for mk in RS_MASKS:
        pl.semaphore_signal(bsem, inc=1, device_id=(my ^ mk,),
                            device_id_type=pl.DeviceIdType.MESH)
    pl.semaphore_wait(bsem, 4)

    start = jnp.int32(0)
    length = SQ
    for k, mk in enumerate(RS_MASKS):
        half = length // 2
        partner = my ^ mk
        upper = (my & mk) != 0
        keep = pl.multiple_of(
            jnp.where(upper, start + half, start).astype(jnp.int32), 64
        )
        give = pl.multiple_of(
            jnp.where(upper, start, start + half).astype(jnp.int32), 64
        )
        sbuf_ref[0:half, :] = acc_ref[pl.ds(give, half), :].astype(jnp.bfloat16)
        step = pltpu.make_async_remote_copy(
            src_ref=sbuf_ref.at[0:half, :],
            dst_ref=rbuf_ref.at[k, 0:half, :],
            send_sem=rs_send_sems.at[k],
            recv_sem=rs_recv_sems.at[k],
            device_id=(partner,),
            device_id_type=pl.DeviceIdType.MESH,
        )
        step.start()
        step.wait_send()
        step.wait_recv()
        acc_ref[pl.ds(keep, half), :] = (
            acc_ref[pl.ds(keep, half), :]
            + rbuf_ref[k, 0:half, :].astype(jnp.float32)
        )
        start = keep
        length = half

    gbuf_ref[pl.ds(start, 64), :] = acc_ref[pl.ds(start, 64), :].astype(
        jnp.bfloat16
    )
    cur_start = start
    cur_len = 64
    for k, mk in enumerate(DB_MASKS):
        partner = my ^ mk
        bit = (my & mk) != 0
        partner_start = pl.multiple_of(
            jnp.where(
                bit, cur_start - CONTRIB[mk], cur_start + CONTRIB[mk]
            ).astype(jnp.int32),
            64,
        )
        send = pltpu.make_async_remote_copy(
            src_ref=gbuf_ref.at[pl.ds(cur_start, cur_len), :],
            dst_ref=gbuf_ref.at[pl.ds(cur_start, cur_len), :],
            send_sem=db_send_sems.at[k],
            recv_sem=db_recv_sems.at[k],
            device_id=(partner,),
            device_id_type=pl.DeviceIdType.MESH,
        )
        send.start()
        send.wait_send()
        recv = pltpu.make_async_remote_copy(
            src_ref=gbuf_ref.at[pl.ds(partner_start, cur_len), :],
            dst_ref=gbuf_ref.at[pl.ds(partner_start, cur_len), :],
            send_sem=db_send_sems.at[k],
            recv_sem=db_recv_sems.at[k],
            device_id=(partner,),
            device_id_type=pl.DeviceIdType.MESH,
        )
        recv.wait_recv()
        cur_start = pl.multiple_of(
            jnp.minimum(cur_start, partner_start), 64
        )
        cur_len *= 2

    out_ref[...] = gbuf_ref[...].astype(jnp.float32)


def kernel(x, Wq, K_ext, V_ext, Wo):
    pos = lax.axis_index("i")
    xb = x[0].astype(jnp.bfloat16)
    wq = Wq.astype(jnp.bfloat16)
    wo = Wo.astype(jnp.bfloat16)
    kh = lax.dynamic_slice(
        K_ext, (0, 0, pos * HEADS_PER_SHARD, 0), (1, SKV, HEADS_PER_SHARD, DH)
    )[0]
    vh = lax.dynamic_slice(
        V_ext, (0, 0, pos * HEADS_PER_SHARD, 0), (1, SKV, HEADS_PER_SHARD, DH)
    )[0]
    kh = jnp.transpose(kh, (1, 0, 2)).astype(jnp.bfloat16)
    vh = jnp.transpose(vh, (1, 0, 2)).astype(jnp.bfloat16)

    out = pl.pallas_call(
        _body,
        out_shape=jax.ShapeDtypeStruct((SQ, D_MODEL), jnp.float32),
        in_specs=[pl.BlockSpec(memory_space=pltpu.VMEM)] * 5,
        out_specs=pl.BlockSpec(memory_space=pltpu.VMEM),
        scratch_shapes=[
            pltpu.VMEM((SQ, D_MODEL), jnp.bfloat16),
            pltpu.VMEM((SQ, D_MODEL), jnp.bfloat16),
            pltpu.VMEM((SQ, D_MODEL), jnp.float32),
            pltpu.VMEM((SQ // 2, D_MODEL), jnp.bfloat16),
            pltpu.VMEM((4, SQ // 2, D_MODEL), jnp.bfloat16),
            pltpu.VMEM((SQ, D_MODEL), jnp.bfloat16),
            pltpu.SemaphoreType.DMA((4,)),
            pltpu.SemaphoreType.DMA((4,)),
            pltpu.SemaphoreType.DMA((4,)),
            pltpu.SemaphoreType.DMA((4,)),
        ],
        compiler_params=pltpu.CompilerParams(collective_id=0),
    )(xb, wq, kh, vh, wo)
    return out.reshape(1, SQ, D_MODEL)
